# baseline (speedup 1.0000x reference)
import functools

import jax
import jax.numpy as jnp
from jax import lax
from jax.experimental import pallas as pl
from jax.experimental.pallas import tpu as pltpu
from jax.experimental.pallas import tpu_sc as plsc

NUM_EMB = 1_000_000
DIM = 64
B = 4096
S = 200
NW = 32
CH = 128
L = 16

_mesh = plsc.VectorSubcoreMesh(core_axis_name="c", subcore_axis_name="s")


@functools.partial(
    pl.kernel,
    mesh=_mesh,
    out_type=jax.ShapeDtypeStruct((S, DIM, B), jnp.float32),
    scratch_types=(
        [
            pltpu.VMEM((S, CH), jnp.int32),
            pltpu.VMEM((CH, DIM), jnp.float32),
            pltpu.VMEM((CH, DIM), jnp.float32),
            pltpu.VMEM((DIM, CH), jnp.float32),
            pltpu.VMEM((DIM, CH), jnp.float32),
        ]
        + [pltpu.SemaphoreType.DMA for _ in range(4)]
    ),
    compiler_params=pltpu.CompilerParams(
        use_tc_tiling_on_sc=False, needs_layout_passes=False
    ),
)
def _gather_kernel(idx_hbm, w_hbm, out_hbm, idx_v, g0, g1, o0, o1,
                   gs0, gs1, ss0, ss1):
    gbuf = (g0, g1)
    obuf = (o0, o1)
    gsem = (gs0, gs1)
    ssem = (ss0, ss1)

    wid = lax.axis_index("s") * 2 + lax.axis_index("c")
    b0 = wid * CH

    pltpu.sync_copy(idx_hbm.at[wid], idx_v)

    def start_gather(k, j):
        pltpu.async_copy(w_hbm.at[idx_v.at[j]], gbuf[k], gsem[k])

    def wait_gather(k):
        pltpu.make_async_copy(
            w_hbm.at[idx_v.at[0]], gbuf[k], gsem[k]
        ).wait()

    def start_scatter(k, j):
        pltpu.async_copy(
            obuf[k], out_hbm.at[j, :, pl.ds(b0, CH)], ssem[k]
        )

    def wait_scatter(k):
        pltpu.make_async_copy(
            obuf[k], out_hbm.at[0, :, pl.ds(b0, CH)], ssem[k]
        ).wait()

    rows = [lax.iota(jnp.int32, L) + (L * gi) for gi in range(CH // L)]

    def transpose(k):
        def body(c, cvec):
            for gi in range(CH // L):
                v = plsc.load_gather(gbuf[k], [rows[gi], cvec])
                plsc.store_scatter(obuf[k], [cvec, rows[gi]], v)
            return cvec + 1

        lax.fori_loop(0, DIM, body, jnp.zeros((L,), jnp.int32),
                      unroll=False)

    start_gather(0, 0)
    start_gather(1, 1)

    def step(j, k):
        wait_scatter(k)
        wait_gather(k)
        transpose(k)
        start_scatter(k, j)

    for j in range(2):
        wait_gather(j)
        transpose(j)
        start_scatter(j, j)
        start_gather(j, j + 2)

    def group_body(g, carry):
        for k in range(2):
            j = 2 * g + k
            step(j, k)
            start_gather(k, j + 2)
        return carry

    lax.fori_loop(1, S // 2 - 1, group_body, 0, unroll=False)

    for k in range(2):
        step(S - 2 + k, k)
    for k in range(2):
        wait_scatter(k)


def kernel(token_ids, W):
    idx = token_ids.astype(jnp.int32)
    idx3 = jnp.transpose(idx.reshape(NW, CH, S), (0, 2, 1))
    out = _gather_kernel(idx3, W)
    return jnp.transpose(out, (2, 0, 1))


# speedup vs baseline: 1.5952x; 1.5952x over previous
import functools

import jax
import jax.numpy as jnp
from jax import lax
from jax.experimental import pallas as pl
from jax.experimental.pallas import tpu as pltpu
from jax.experimental.pallas import tpu_sc as plsc

NUM_EMB = 1_000_000
DIM = 64
B = 4096
S = 200
NW = 32
CH = 128
L = 16
NG = 4
NO = 2
OST = CH + 1

_mesh = plsc.VectorSubcoreMesh(core_axis_name="c", subcore_axis_name="s")


@functools.partial(
    pl.kernel,
    mesh=_mesh,
    out_type=jax.ShapeDtypeStruct((S, DIM, B), jnp.float32),
    scratch_types=(
        [
            pltpu.VMEM((CH, S), jnp.int32),
            pltpu.VMEM((S, CH), jnp.int32),
        ]
        + [pltpu.VMEM((CH, DIM), jnp.float32) for _ in range(NG)]
        + [pltpu.VMEM((DIM, OST), jnp.float32) for _ in range(NO)]
        + [pltpu.SemaphoreType.DMA for _ in range(NG + NO)]
    ),
    compiler_params=pltpu.CompilerParams(
        use_tc_tiling_on_sc=False, needs_layout_passes=False
    ),
)
def _gather_kernel(idx_hbm, w_hbm, out_hbm, idxT_v, idx_v, *rest):
    gbuf = rest[:NG]
    obuf = rest[NG:NG + NO]
    gsem = rest[NG + NO:2 * NG + NO]
    ssem = rest[2 * NG + NO:]

    wid = lax.axis_index("s") * 2 + lax.axis_index("c")
    b0 = wid * CH

    pltpu.sync_copy(idx_hbm.at[wid], idxT_v)

    rows = [lax.iota(jnp.int32, L) + (L * gi) for gi in range(CH // L)]
    cvecs = [lax.iota(jnp.int32, L) + (L * kk) for kk in range(DIM // L)]

    def idx_body(j, jvec):
        for gi in range(CH // L):
            v = plsc.load_gather(idxT_v, [rows[gi], jvec])
            idx_v[j, pl.ds(L * gi, L)] = v
        return jvec + 1

    lax.fori_loop(0, S, idx_body, jnp.zeros((L,), jnp.int32), unroll=False)

    def start_gather(k, j):
        pltpu.async_copy(w_hbm.at[idx_v.at[j]], gbuf[k], gsem[k])

    def wait_gather(k):
        pltpu.make_async_copy(
            w_hbm.at[idx_v.at[0]], gbuf[k], gsem[k]
        ).wait()

    def start_scatter(k, j):
        pltpu.async_copy(
            obuf[k].at[:, pl.ds(0, CH)],
            out_hbm.at[j, :, pl.ds(b0, CH)],
            ssem[k],
        )

    def wait_scatter(k):
        pltpu.make_async_copy(
            obuf[k].at[:, pl.ds(0, CH)],
            out_hbm.at[0, :, pl.ds(b0, CH)],
            ssem[k],
        ).wait()

    def transpose(kg, ko):
        def body(i, ivec):
            row = gbuf[kg].at[i]
            for kk in range(DIM // L):
                v = row[pl.ds(L * kk, L)]
                plsc.store_scatter(obuf[ko], [cvecs[kk], ivec], v)
            return ivec + 1

        lax.fori_loop(0, CH, body, jnp.zeros((L,), jnp.int32),
                      unroll=False)

    for j in range(NG):
        start_gather(j, j)

    def step(j, kg, ko, swait, gissue):
        if swait:
            wait_scatter(ko)
        wait_gather(kg)
        transpose(kg, ko)
        start_scatter(ko, j)
        if gissue:
            start_gather(kg, j + NG)

    for j in range(NO):
        step(j, j % NG, j % NO, False, True)
    for j in range(NO, NG):
        step(j, j % NG, j % NO, True, True)

    def group_body(g, carry):
        for k in range(NG):
            step(g * NG + k, k, k % NO, True, True)
        return carry

    lax.fori_loop(1, S // NG - 1, group_body, 0, unroll=False)

    for j in range(S - NG, S):
        step(j, j % NG, j % NO, True, False)
    for k in range(NO):
        wait_scatter(k)


def kernel(token_ids, W):
    idx3 = token_ids.astype(jnp.int32).reshape(NW, CH, S)
    out = _gather_kernel(idx3, W)
    return jnp.transpose(out, (2, 0, 1))
